# baseline (speedup 1.0000x reference)
import functools

import jax
import jax.numpy as jnp
from jax import lax
from jax.experimental import pallas as pl
from jax.experimental.pallas import tpu as pltpu
from jax.experimental.pallas import tpu_sc as plsc

N = 10000
E = 320000
D = 128
BN_EPS = 1e-5

NUM_TILES = 16
CHUNK = 128
NBUF = 3
NCHUNK = 159
NGROUP = NCHUNK // NBUF
TILE_EDGES = NCHUNK * CHUNK
E_PAD = NUM_TILES * TILE_EDGES
N_PAD = N + 8
ROWS_PER_TILE = 624
ROWS_TAIL = N - NUM_TILES * ROWS_PER_TILE


def _sc_aggregate(x_pad, edges):
    mesh = plsc.VectorSubcoreMesh(core_axis_name="c", subcore_axis_name="s")

    @functools.partial(
        pl.kernel,
        out_type=jax.ShapeDtypeStruct((2, N, D), jnp.float32),
        mesh=mesh,
        scratch_types=[
            pltpu.VMEM_SHARED((N_PAD, D), jnp.float32),
            pltpu.VMEM((NBUF, 2, CHUNK), jnp.int32),
            pltpu.VMEM((NBUF, CHUNK, D), jnp.float32),
            pltpu.SemaphoreType.DMA((NBUF,)),
            pltpu.SemaphoreType.DMA((NBUF,)),
            pltpu.SemaphoreType.DMA((NBUF,)),
        ],
    )
    def agg_kernel(x_hbm, edges_hbm, out_hbm, acc, idx, rows, isem, gsem, ssem):
        c = lax.axis_index("c")
        s = lax.axis_index("s")
        r0 = pl.multiple_of(s * ROWS_PER_TILE, 8)

        def idx_start(j, b):
            pltpu.async_copy(edges_hbm.at[c, s, j], idx.at[b], isem.at[b])

        def idx_wait(j, b):
            pltpu.make_async_copy(
                edges_hbm.at[c, s, j], idx.at[b], isem.at[b]).wait()

        def gather_start(b):
            pltpu.async_copy(x_hbm.at[idx.at[b, 0]], rows.at[b], gsem.at[b])

        def gather_wait(b):
            pltpu.make_async_copy(
                x_hbm.at[idx.at[b, 0]], rows.at[b], gsem.at[b]).wait()

        def scatter_start(b):
            pltpu.async_copy(
                rows.at[b], acc.at[idx.at[b, 1]], ssem.at[b], add=True)

        def scatter_wait(b):
            pltpu.make_async_copy(
                rows.at[b], acc.at[idx.at[b, 1]], ssem.at[b]).wait()

        for b in range(NBUF):
            idx_start(b, b)
        pltpu.sync_copy(x_hbm.at[pl.ds(r0, ROWS_PER_TILE)],
                        acc.at[pl.ds(r0, ROWS_PER_TILE)])

        @pl.when(s == NUM_TILES - 1)
        def _init_tail():
            t0 = NUM_TILES * ROWS_PER_TILE
            pltpu.sync_copy(x_hbm.at[pl.ds(t0, N_PAD - t0)],
                            acc.at[pl.ds(t0, N_PAD - t0)])

        plsc.subcore_barrier()

        def group(g, last):
            for b in range(NBUF):
                idx_wait(g * NBUF + b, b)
                gather_start(b)
            for b in range(NBUF):
                gather_wait(b)
                scatter_start(b)
            for b in range(NBUF):
                scatter_wait(b)
                if not last:
                    idx_start((g + 1) * NBUF + b, b)

        def outer(g, carry):
            group(g, last=False)
            return carry

        lax.fori_loop(0, NGROUP - 1, outer, 0)
        group(NGROUP - 1, last=True)

        plsc.subcore_barrier()

        pltpu.sync_copy(acc.at[pl.ds(r0, ROWS_PER_TILE)],
                        out_hbm.at[c, pl.ds(r0, ROWS_PER_TILE)])

        @pl.when(s == NUM_TILES - 1)
        def _out_tail():
            t0 = NUM_TILES * ROWS_PER_TILE
            pltpu.sync_copy(acc.at[pl.ds(t0, ROWS_TAIL)],
                            out_hbm.at[c, pl.ds(t0, ROWS_TAIL)])

    return agg_kernel(x_pad, edges)


def _tc_mlp(pre, w1t0, w2t0, g10, b10, g20, b20, w1t1, w2t1, g11, b11, g21, b21):
    def body(pre_ref, w1t0_r, w2t0_r, g10_r, b10_r, g20_r, b20_r,
             w1t1_r, w2t1_r, g11_r, b11_r, g21_r, b21_r, out_ref):
        def bn_relu(h, g, b):
            mean = jnp.mean(h, axis=0, keepdims=True)
            var = jnp.mean((h - mean) * (h - mean), axis=0, keepdims=True)
            return jnp.maximum((h - mean) * lax.rsqrt(var + BN_EPS) * g + b, 0.0)

        def rel(p, w1t, w2t, g1, b1, g2, b2):
            h = jnp.dot(p, w1t, preferred_element_type=jnp.float32)
            h = bn_relu(h, g1, b1)
            h = jnp.dot(h, w2t, preferred_element_type=jnp.float32)
            return bn_relu(h, g2, b2)

        out_ref[...] = (
            rel(pre_ref[0], w1t0_r[...], w2t0_r[...], g10_r[...], b10_r[...],
                g20_r[...], b20_r[...])
            + rel(pre_ref[1], w1t1_r[...], w2t1_r[...], g11_r[...], b11_r[...],
                  g21_r[...], b21_r[...]))

    return pl.pallas_call(
        body,
        out_shape=jax.ShapeDtypeStruct((N, D), jnp.float32),
    )(pre, w1t0, w2t0, g10, b10, g20, b20, w1t1, w2t1, g11, b11, g21, b21)


def _prep_edges(edge_index):
    pad = E_PAD - E
    src = jnp.concatenate([edge_index[0], jnp.full((pad,), N, jnp.int32)])
    dst = jnp.concatenate([edge_index[1], jnp.full((pad,), N, jnp.int32)])
    src = src.reshape(NUM_TILES, NCHUNK, 1, CHUNK)
    dst = dst.reshape(NUM_TILES, NCHUNK, 1, CHUNK)
    return jnp.concatenate([src, dst], axis=2)


@jax.jit
def kernel(x, edge_index_rel0, edge_index_rel1,
           W1_0, W2_0, g1_0, b1_0, g2_0, b2_0,
           W1_1, W2_1, g1_1, b1_1, g2_1, b2_1):
    edges = jnp.stack([_prep_edges(edge_index_rel0),
                       _prep_edges(edge_index_rel1)])
    x_pad = jnp.concatenate([x, jnp.zeros((N_PAD - N, D), jnp.float32)])
    pre = _sc_aggregate(x_pad, edges)
    row = lambda v: v.reshape(1, D)
    return _tc_mlp(pre,
                   W1_0.T, W2_0.T, row(g1_0), row(b1_0), row(g2_0), row(b2_0),
                   W1_1.T, W2_1.T, row(g1_1), row(b1_1), row(g2_1), row(b2_1))
